# baseline (speedup 1.0000x reference)
import functools

import jax
import jax.numpy as jnp
from jax import lax
from jax.experimental import pallas as pl
from jax.experimental.pallas import tpu as pltpu
from jax.experimental.pallas import tpu_sc as plsc

_N = 100000
_G = 1024
_NC = 2
_NS = 16
_HALF = _G // _NC
_LANES = 16
_S = 395
_CHUNK = _LANES * _S
_TAIL = _N - 15 * _CHUNK
_S_TAIL = _TAIL // _LANES

_mesh = plsc.VectorSubcoreMesh(core_axis_name="c", subcore_axis_name="s")


@functools.partial(
    pl.kernel,
    mesh=_mesh,
    compiler_params=pltpu.CompilerParams(
        needs_layout_passes=False,
        disable_bounds_checks=True,
        disable_semaphore_checks=True,
    ),
    out_type=jax.ShapeDtypeStruct((_G,), jnp.float32),
    scratch_types=[
        pltpu.VMEM((_CHUNK,), jnp.float32),
        pltpu.VMEM((_CHUNK,), jnp.int32),
        pltpu.VMEM((_HALF,), jnp.float32),
        pltpu.VMEM((_HALF,), jnp.int32),
        pltpu.VMEM_SHARED((_HALF,), jnp.float32),
        pltpu.SemaphoreType.DMA,
        pltpu.SemaphoreType.DMA,
        pltpu.SemaphoreType.DMA,
        pltpu.SemaphoreType.DMA,
    ],
)
def _segment_sum_sc(
    energy_hbm, idx_hbm, out_hbm, e_v, i_v, acc_v, iota_v, shared,
    sem_e, sem_i, sem_e2, sem_i2
):
    cid = lax.axis_index("c")
    sid = lax.axis_index("s")
    base = sid * _CHUNK

    cp_e = pltpu.async_copy(
        energy_hbm.at[pl.ds(base, _TAIL)], e_v.at[pl.ds(0, _TAIL)], sem_e
    )
    cp_i = pltpu.async_copy(
        idx_hbm.at[pl.ds(base, _TAIL)], i_v.at[pl.ds(0, _TAIL)], sem_i
    )
    base2 = jnp.minimum(base + _TAIL, _N - (_CHUNK - _TAIL)).astype(jnp.int32)
    cp_e2 = pltpu.async_copy(
        energy_hbm.at[pl.ds(base2, _CHUNK - _TAIL)],
        e_v.at[pl.ds(_TAIL, _CHUNK - _TAIL)],
        sem_e2,
    )
    cp_i2 = pltpu.async_copy(
        idx_hbm.at[pl.ds(base2, _CHUNK - _TAIL)],
        i_v.at[pl.ds(_TAIL, _CHUNK - _TAIL)],
        sem_i2,
    )

    zeros = jnp.zeros((_LANES,), jnp.float32)
    iota16 = lax.iota(jnp.int32, _LANES)

    def init_body(j, carry):
        acc_v[pl.ds(j * _LANES, _LANES)] = zeros
        iota_v[pl.ds(j * _LANES, _LANES)] = iota16 + j * _LANES
        return carry

    lax.fori_loop(0, _HALF // _LANES, init_body, 0)

    @pl.when(sid == 0)
    def _():
        pltpu.sync_copy(acc_v, shared)

    plsc.subcore_barrier()

    lo = cid * _HALF

    cp_e.wait()
    cp_i.wait()
    lane_base_a = iota16 * _S_TAIL

    @plsc.parallel_loop(0, _S_TAIL, 1, unroll=8)
    def _(j):
        node = lane_base_a + j
        e = plsc.load_gather(e_v, [node])
        ix = plsc.load_gather(i_v, [node]) - lo
        mask = plsc.bitcast(ix, jnp.uint32) < jnp.uint32(_HALF)
        plsc.addupdate_scatter(acc_v, [ix], e, mask=mask)

    cp_e2.wait()
    cp_i2.wait()
    _S_B = ((_CHUNK - _TAIL) // _LANES) | 1
    lane_base_b = _TAIL + iota16 * _S_B
    n_b = jnp.where(sid == _NS - 1, 0, _S_B).astype(jnp.int32)

    @plsc.parallel_loop(0, n_b, 1, unroll=8)
    def _(j):
        node = lane_base_b + j
        node_c = jnp.minimum(node, _CHUNK - 1)
        e = plsc.load_gather(e_v, [node_c])
        ix = plsc.load_gather(i_v, [node_c]) - lo
        mask = (node < _CHUNK) & (
            plsc.bitcast(ix, jnp.uint32) < jnp.uint32(_HALF)
        )
        plsc.addupdate_scatter(acc_v, [ix], e, mask=mask)

    pltpu.sync_copy(acc_v, shared.at[iota_v], add=True)
    plsc.subcore_barrier()

    @pl.when(sid == 0)
    def _():
        pltpu.sync_copy(shared, out_hbm.at[pl.ds(lo, _HALF)])


def kernel(node_energy, batch, num_graphs):
    del num_graphs
    return _segment_sum_sc(
        node_energy.astype(jnp.float32), batch.astype(jnp.int32)
    )


# speedup vs baseline: 1.0126x; 1.0126x over previous
import functools

import jax
import jax.numpy as jnp
from jax import lax
from jax.experimental import pallas as pl
from jax.experimental.pallas import tpu as pltpu
from jax.experimental.pallas import tpu_sc as plsc

_N = 100000
_G = 1024
_NC = 2
_NS = 16
_HALF = _G // _NC
_LANES = 16
_S = 395
_CHUNK = _LANES * _S
_TAIL = _N - 15 * _CHUNK
_S_TAIL = _TAIL // _LANES

_mesh = plsc.VectorSubcoreMesh(core_axis_name="c", subcore_axis_name="s")


@functools.partial(
    pl.kernel,
    mesh=_mesh,
    compiler_params=pltpu.CompilerParams(
        needs_layout_passes=False,
        disable_bounds_checks=True,
        disable_semaphore_checks=True,
    ),
    out_type=jax.ShapeDtypeStruct((_G,), jnp.float32),
    scratch_types=[
        pltpu.VMEM((_CHUNK,), jnp.float32),
        pltpu.VMEM((_CHUNK,), jnp.int32),
        pltpu.VMEM((_HALF,), jnp.float32),
        pltpu.VMEM((_HALF,), jnp.int32),
        pltpu.VMEM_SHARED((_HALF,), jnp.float32),
        pltpu.SemaphoreType.DMA,
        pltpu.SemaphoreType.DMA,
    ],
)
def _segment_sum_sc(
    energy_hbm, idx_hbm, out_hbm, e_v, i_v, acc_v, iota_v, shared, sem_e, sem_i
):
    cid = lax.axis_index("c")
    sid = lax.axis_index("s")
    base = sid * _CHUNK

    cp_e = pltpu.async_copy(
        energy_hbm.at[pl.ds(base, _TAIL)], e_v.at[pl.ds(0, _TAIL)], sem_e
    )
    cp_i = pltpu.async_copy(
        idx_hbm.at[pl.ds(base, _TAIL)], i_v.at[pl.ds(0, _TAIL)], sem_i
    )
    base2 = jnp.minimum(base + _TAIL, _N - (_CHUNK - _TAIL)).astype(jnp.int32)
    cp_e2 = pltpu.async_copy(
        energy_hbm.at[pl.ds(base2, _CHUNK - _TAIL)],
        e_v.at[pl.ds(_TAIL, _CHUNK - _TAIL)],
        sem_e,
    )
    cp_i2 = pltpu.async_copy(
        idx_hbm.at[pl.ds(base2, _CHUNK - _TAIL)],
        i_v.at[pl.ds(_TAIL, _CHUNK - _TAIL)],
        sem_i,
    )

    zeros = jnp.zeros((_LANES,), jnp.float32)
    iota16 = lax.iota(jnp.int32, _LANES)

    def init_body(j, carry):
        acc_v[pl.ds(j * _LANES, _LANES)] = zeros
        iota_v[pl.ds(j * _LANES, _LANES)] = iota16 + j * _LANES
        return carry

    lax.fori_loop(0, _HALF // _LANES, init_body, 0)

    @pl.when(sid == 0)
    def _():
        pltpu.sync_copy(acc_v, shared)

    plsc.subcore_barrier()

    cp_e.wait()
    cp_i.wait()
    cp_e2.wait()
    cp_i2.wait()

    lo = cid * _HALF
    s_run = jnp.where(sid == _NS - 1, _S_TAIL, _S).astype(jnp.int32)
    lane_base = iota16 * s_run

    @plsc.parallel_loop(0, s_run, 1, unroll=8)
    def _(j):
        node = lane_base + j
        e = plsc.load_gather(e_v, [node])
        ix = plsc.load_gather(i_v, [node]) - lo
        mask = plsc.bitcast(ix, jnp.uint32) < jnp.uint32(_HALF)
        plsc.addupdate_scatter(acc_v, [ix], e, mask=mask)

    pltpu.sync_copy(acc_v, shared.at[iota_v], add=True)
    plsc.subcore_barrier()

    @pl.when(sid == 0)
    def _():
        pltpu.sync_copy(shared, out_hbm.at[pl.ds(lo, _HALF)])


def kernel(node_energy, batch, num_graphs):
    del num_graphs
    return _segment_sum_sc(
        node_energy.astype(jnp.float32), batch.astype(jnp.int32)
    )
